# baseline (speedup 1.0000x reference)
import functools

import jax
import jax.numpy as jnp
from jax import lax
from jax.experimental import pallas as pl
from jax.experimental.pallas import tpu as pltpu
from jax.experimental.pallas import tpu_sc as plsc

B, T = 8, 40
C, H, W = 3, 192, 192
NC, NS, L = 2, 16, 16
NW = NC * NS
HH = H // 2
NCH = 2 * C
ROW_UNROLL = W // L
SC_NUM, SC_DEN = 13, 32


def _split_point(nv):
    return (nv * SC_NUM) // SC_DEN


def _sc_chunk_sum(xbuf, ybuf):
    def step(r, acc):
        for u in range(ROW_UNROLL):
            xv = xbuf[r, pl.ds(u * L, L)]
            yv = ybuf[r, pl.ds(u * L, L)]
            acc = acc + jnp.abs(xv - yv)
        return acc

    return lax.fori_loop(0, HH, step, jnp.zeros((L,), jnp.float32))


@functools.partial(
    pl.kernel,
    out_type=jax.ShapeDtypeStruct((NW, L), jnp.float32),
    mesh=plsc.VectorSubcoreMesh(
        core_axis_name="c", subcore_axis_name="s", num_cores=NC, num_subcores=NS
    ),
    compiler_params=pltpu.CompilerParams(needs_layout_passes=False),
    scratch_types=[
        pltpu.VMEM((HH, W), jnp.float32),
        pltpu.VMEM((HH, W), jnp.float32),
        pltpu.VMEM((HH, W), jnp.float32),
        pltpu.VMEM((HH, W), jnp.float32),
        pltpu.VMEM((16,), jnp.int32),
        pltpu.VMEM((L,), jnp.float32),
        pltpu.SemaphoreType.DMA,
        pltpu.SemaphoreType.DMA,
        pltpu.SemaphoreType.DMA,
        pltpu.SemaphoreType.DMA,
    ],
)
def _sc_l1(x_hbm, y_hbm, vl_hbm, out_hbm,
           xb0, xb1, yb0, yb1, vlv, accv, sx0, sx1, sy0, sy1):
    cid = lax.axis_index("c")
    sid = lax.axis_index("s")
    wid = sid * NC + cid

    pltpu.sync_copy(vl_hbm, vlv.at[pl.ds(0, B)])
    iota = lax.iota(jnp.int32, 16)
    vl = jnp.where(iota < B, vlv[...], 0)
    cum = plsc.cumsum(vl)
    cumex = cum - vl
    nv = jnp.max(cum)
    n_sc = _split_point(nv)

    nf = (n_sc - wid + (NW - 1)) // NW
    nf = jnp.maximum(nf, 0)
    nq = nf * NCH

    def chunk_loc(q):
        k = q // NCH
        c = q - k * NCH
        j = wid + NW * k
        bb = jnp.sum((cum <= j).astype(jnp.int32))
        start = jnp.sum(jnp.where(iota == bb, cumex, 0))
        t = j - start
        ch = c // 2
        h0 = (c - 2 * ch) * HH
        return bb, t, ch, h0

    def start_q(q, xbuf, ybuf, sx, sy):
        bb, t, ch, h0 = chunk_loc(q)
        pltpu.async_copy(x_hbm.at[bb, t, ch, pl.ds(h0, HH), :], xbuf, sx)
        pltpu.async_copy(y_hbm.at[bb, t, ch, pl.ds(h0, HH), :], ybuf, sy)

    def wait_q(xbuf, ybuf, sx, sy):
        pltpu.make_async_copy(x_hbm.at[0, 0, 0, pl.ds(0, HH), :], xbuf, sx).wait()
        pltpu.make_async_copy(y_hbm.at[0, 0, 0, pl.ds(0, HH), :], ybuf, sy).wait()

    @pl.when(nq > 0)
    def _():
        start_q(0, xb0, yb0, sx0, sy0)

    @pl.when(nq > 1)
    def _():
        start_q(1, xb1, yb1, sx1, sy1)

    def pair(g, acc):
        q0 = 2 * g
        q1 = q0 + 1
        wait_q(xb0, yb0, sx0, sy0)
        acc = acc + _sc_chunk_sum(xb0, yb0)

        @pl.when(q0 + 2 < nq)
        def _():
            start_q(q0 + 2, xb0, yb0, sx0, sy0)

        @pl.when(q1 < nq)
        def _():
            wait_q(xb1, yb1, sx1, sy1)

        s1 = _sc_chunk_sum(xb1, yb1)
        acc = acc + jnp.where(q1 < nq, s1, 0.0)

        @pl.when(q1 + 2 < nq)
        def _():
            start_q(q1 + 2, xb1, yb1, sx1, sy1)

        return acc

    acc = lax.fori_loop(0, (nq + 1) // 2, pair, jnp.zeros((L,), jnp.float32))
    accv[...] = acc
    pltpu.sync_copy(accv, out_hbm.at[wid])


def _tc_body(vl_ref, x_hbm, y_hbm, out_ref, *bufs_and_sems):
    nb = TC_NBUF
    xbs = bufs_and_sems[0:nb]
    ybs = bufs_and_sems[nb:2 * nb]
    sxs = bufs_and_sems[2 * nb:3 * nb]
    sys_ = bufs_and_sems[3 * nb:4 * nb]
    def cum_scan(i, carry):
        nv, _ = carry
        return nv + vl_ref[i], 0

    nv, _ = lax.fori_loop(0, B, cum_scan, (0, 0))
    n_sc = _split_point(nv)
    n_tc = nv - n_sc

    def frame_of(j):
        def body(i, carry):
            b, cum, start = carry
            newcum = cum + vl_ref[i]
            take = newcum <= j
            b = jnp.where(take, b + 1, b)
            start = jnp.where(take, newcum, start)
            return b, newcum, start

        b, _, start = lax.fori_loop(0, B, body, (0, 0, 0))
        return b, j - start

    def start_q(q, xbuf, ybuf, sx, sy):
        b, t = frame_of(n_sc + q)
        pltpu.make_async_copy(x_hbm.at[b, t], xbuf, sx).start()
        pltpu.make_async_copy(y_hbm.at[b, t], ybuf, sy).start()

    def wait_q(xbuf, ybuf, sx, sy):
        pltpu.make_async_copy(x_hbm.at[0, 0], xbuf, sx).wait()
        pltpu.make_async_copy(y_hbm.at[0, 0], ybuf, sy).wait()

    for k in range(nb):
        @pl.when(n_tc > k)
        def _(k=k):
            start_q(k, xbs[k], ybs[k], sxs[k], sys_[k])

    def plane_sum(xbuf, ybuf):
        d = jnp.abs(xbuf[...] - ybuf[...])
        return d[0] + d[1] + d[2]

    def group(g, acc):
        qbase = nb * g
        wait_q(xbs[0], ybs[0], sxs[0], sys_[0])
        acc = acc + plane_sum(xbs[0], ybs[0])

        @pl.when(qbase + nb < n_tc)
        def _():
            start_q(qbase + nb, xbs[0], ybs[0], sxs[0], sys_[0])

        for k in range(1, nb):
            q = qbase + k

            @pl.when(q < n_tc)
            def _(k=k):
                wait_q(xbs[k], ybs[k], sxs[k], sys_[k])

            sk = plane_sum(xbs[k], ybs[k])
            acc = acc + jnp.where(q < n_tc, sk, jnp.zeros((H, W), jnp.float32))

            @pl.when(q + nb < n_tc)
            def _(k=k, q=q):
                start_q(q + nb, xbs[k], ybs[k], sxs[k], sys_[k])

        return acc

    acc = lax.fori_loop(0, (n_tc + nb - 1) // nb, group,
                        jnp.zeros((H, W), jnp.float32))
    out_ref[0, 0] = jnp.sum(acc)


TC_NBUF = 6

_tc_l1 = pl.pallas_call(
    _tc_body,
    grid=(1,),
    in_specs=[
        pl.BlockSpec(memory_space=pltpu.SMEM),
        pl.BlockSpec(memory_space=pltpu.HBM),
        pl.BlockSpec(memory_space=pltpu.HBM),
    ],
    out_specs=pl.BlockSpec(memory_space=pltpu.SMEM),
    out_shape=jax.ShapeDtypeStruct((1, 1), jnp.float32),
    scratch_shapes=(
        [pltpu.VMEM((C, H, W), jnp.float32) for _ in range(2 * TC_NBUF)]
        + [pltpu.SemaphoreType.DMA for _ in range(2 * TC_NBUF)]
    ),
)


def kernel(inputs, gt, valid_len):
    vl32 = valid_len.astype(jnp.int32)
    sc_partials = _sc_l1(inputs, gt, vl32)
    tc_partial = _tc_l1(vl32, inputs, gt)
    total = jnp.sum(sc_partials) + tc_partial[0, 0]
    count = jnp.sum(valid_len).astype(inputs.dtype) * (C * H * W)
    return total / count


# speedup vs baseline: 1.0190x; 1.0190x over previous
import functools

import jax
import jax.numpy as jnp
from jax import lax
from jax.experimental import pallas as pl
from jax.experimental.pallas import tpu as pltpu
from jax.experimental.pallas import tpu_sc as plsc

B, T = 8, 40
C, H, W = 3, 192, 192
NC, NS, L = 2, 16, 16
NW = NC * NS
HH = H // 2
NCH = 2 * C
ROW_UNROLL = W // L
SC_NUM, SC_DEN = 10, 32


def _split_point(nv):
    return (nv * SC_NUM) // SC_DEN


def _sc_chunk_sum(xbuf, ybuf):
    def step(r, acc):
        for u in range(ROW_UNROLL):
            xv = xbuf[r, pl.ds(u * L, L)]
            yv = ybuf[r, pl.ds(u * L, L)]
            acc = acc + jnp.abs(xv - yv)
        return acc

    return lax.fori_loop(0, HH, step, jnp.zeros((L,), jnp.float32))


@functools.partial(
    pl.kernel,
    out_type=jax.ShapeDtypeStruct((NW, L), jnp.float32),
    mesh=plsc.VectorSubcoreMesh(
        core_axis_name="c", subcore_axis_name="s", num_cores=NC, num_subcores=NS
    ),
    compiler_params=pltpu.CompilerParams(needs_layout_passes=False),
    scratch_types=[
        pltpu.VMEM((HH, W), jnp.float32),
        pltpu.VMEM((HH, W), jnp.float32),
        pltpu.VMEM((HH, W), jnp.float32),
        pltpu.VMEM((HH, W), jnp.float32),
        pltpu.VMEM((16,), jnp.int32),
        pltpu.VMEM((L,), jnp.float32),
        pltpu.SemaphoreType.DMA,
        pltpu.SemaphoreType.DMA,
        pltpu.SemaphoreType.DMA,
        pltpu.SemaphoreType.DMA,
    ],
)
def _sc_l1(x_hbm, y_hbm, vl_hbm, out_hbm,
           xb0, xb1, yb0, yb1, vlv, accv, sx0, sx1, sy0, sy1):
    cid = lax.axis_index("c")
    sid = lax.axis_index("s")
    wid = sid * NC + cid

    pltpu.sync_copy(vl_hbm, vlv.at[pl.ds(0, B)])
    iota = lax.iota(jnp.int32, 16)
    vl = jnp.where(iota < B, vlv[...], 0)
    cum = plsc.cumsum(vl)
    cumex = cum - vl
    nv = jnp.max(cum)
    n_sc = _split_point(nv)

    nf = (n_sc - wid + (NW - 1)) // NW
    nf = jnp.maximum(nf, 0)
    nq = nf * NCH

    def chunk_loc(q):
        k = q // NCH
        c = q - k * NCH
        j = wid + NW * k
        bb = jnp.sum((cum <= j).astype(jnp.int32))
        start = jnp.sum(jnp.where(iota == bb, cumex, 0))
        t = j - start
        ch = c // 2
        h0 = (c - 2 * ch) * HH
        return bb, t, ch, h0

    def start_q(q, xbuf, ybuf, sx, sy):
        bb, t, ch, h0 = chunk_loc(q)
        pltpu.async_copy(x_hbm.at[bb, t, ch, pl.ds(h0, HH), :], xbuf, sx)
        pltpu.async_copy(y_hbm.at[bb, t, ch, pl.ds(h0, HH), :], ybuf, sy)

    def wait_q(xbuf, ybuf, sx, sy):
        pltpu.make_async_copy(x_hbm.at[0, 0, 0, pl.ds(0, HH), :], xbuf, sx).wait()
        pltpu.make_async_copy(y_hbm.at[0, 0, 0, pl.ds(0, HH), :], ybuf, sy).wait()

    @pl.when(nq > 0)
    def _():
        start_q(0, xb0, yb0, sx0, sy0)

    @pl.when(nq > 1)
    def _():
        start_q(1, xb1, yb1, sx1, sy1)

    def pair(g, acc):
        q0 = 2 * g
        q1 = q0 + 1
        wait_q(xb0, yb0, sx0, sy0)
        acc = acc + _sc_chunk_sum(xb0, yb0)

        @pl.when(q0 + 2 < nq)
        def _():
            start_q(q0 + 2, xb0, yb0, sx0, sy0)

        @pl.when(q1 < nq)
        def _():
            wait_q(xb1, yb1, sx1, sy1)

        s1 = _sc_chunk_sum(xb1, yb1)
        acc = acc + jnp.where(q1 < nq, s1, 0.0)

        @pl.when(q1 + 2 < nq)
        def _():
            start_q(q1 + 2, xb1, yb1, sx1, sy1)

        return acc

    acc = lax.fori_loop(0, (nq + 1) // 2, pair, jnp.zeros((L,), jnp.float32))
    accv[...] = acc
    pltpu.sync_copy(accv, out_hbm.at[wid])


def _tc_body(vl_ref, x_hbm, y_hbm, out_ref, *bufs_and_sems):
    nb = TC_NBUF
    xbs = bufs_and_sems[0:nb]
    ybs = bufs_and_sems[nb:2 * nb]
    sxs = bufs_and_sems[2 * nb:3 * nb]
    sys_ = bufs_and_sems[3 * nb:4 * nb]
    def cum_scan(i, carry):
        nv, _ = carry
        return nv + vl_ref[i], 0

    nv, _ = lax.fori_loop(0, B, cum_scan, (0, 0))
    n_sc = _split_point(nv)
    n_tc = nv - n_sc

    def frame_of(j):
        def body(i, carry):
            b, cum, start = carry
            newcum = cum + vl_ref[i]
            take = newcum <= j
            b = jnp.where(take, b + 1, b)
            start = jnp.where(take, newcum, start)
            return b, newcum, start

        b, _, start = lax.fori_loop(0, B, body, (0, 0, 0))
        return b, j - start

    def start_q(q, xbuf, ybuf, sx, sy):
        b, t = frame_of(n_sc + q)
        pltpu.make_async_copy(x_hbm.at[b, t], xbuf, sx).start()
        pltpu.make_async_copy(y_hbm.at[b, t], ybuf, sy).start()

    def wait_q(xbuf, ybuf, sx, sy):
        pltpu.make_async_copy(x_hbm.at[0, 0], xbuf, sx).wait()
        pltpu.make_async_copy(y_hbm.at[0, 0], ybuf, sy).wait()

    for k in range(nb):
        @pl.when(n_tc > k)
        def _(k=k):
            start_q(k, xbs[k], ybs[k], sxs[k], sys_[k])

    def plane_sum(xbuf, ybuf):
        d = jnp.abs(xbuf[...] - ybuf[...])
        return d[0] + d[1] + d[2]

    def group(g, acc):
        qbase = nb * g
        wait_q(xbs[0], ybs[0], sxs[0], sys_[0])
        acc = acc + plane_sum(xbs[0], ybs[0])

        @pl.when(qbase + nb < n_tc)
        def _():
            start_q(qbase + nb, xbs[0], ybs[0], sxs[0], sys_[0])

        for k in range(1, nb):
            q = qbase + k

            @pl.when(q < n_tc)
            def _(k=k):
                wait_q(xbs[k], ybs[k], sxs[k], sys_[k])

            sk = plane_sum(xbs[k], ybs[k])
            acc = acc + jnp.where(q < n_tc, sk, jnp.zeros((H, W), jnp.float32))

            @pl.when(q + nb < n_tc)
            def _(k=k, q=q):
                start_q(q + nb, xbs[k], ybs[k], sxs[k], sys_[k])

        return acc

    acc = lax.fori_loop(0, (n_tc + nb - 1) // nb, group,
                        jnp.zeros((H, W), jnp.float32))
    out_ref[0, 0] = jnp.sum(acc)


TC_NBUF = 6

_tc_l1 = pl.pallas_call(
    _tc_body,
    grid=(1,),
    in_specs=[
        pl.BlockSpec(memory_space=pltpu.SMEM),
        pl.BlockSpec(memory_space=pltpu.HBM),
        pl.BlockSpec(memory_space=pltpu.HBM),
    ],
    out_specs=pl.BlockSpec(memory_space=pltpu.SMEM),
    out_shape=jax.ShapeDtypeStruct((1, 1), jnp.float32),
    scratch_shapes=(
        [pltpu.VMEM((C, H, W), jnp.float32) for _ in range(2 * TC_NBUF)]
        + [pltpu.SemaphoreType.DMA for _ in range(2 * TC_NBUF)]
    ),
)


def kernel(inputs, gt, valid_len):
    vl32 = valid_len.astype(jnp.int32)
    sc_partials = _sc_l1(inputs, gt, vl32)
    tc_partial = _tc_l1(vl32, inputs, gt)
    total = jnp.sum(sc_partials) + tc_partial[0, 0]
    count = jnp.sum(valid_len).astype(inputs.dtype) * (C * H * W)
    return total / count


# speedup vs baseline: 1.0239x; 1.0049x over previous
import functools

import jax
import jax.numpy as jnp
from jax import lax
from jax.experimental import pallas as pl
from jax.experimental.pallas import tpu as pltpu
from jax.experimental.pallas import tpu_sc as plsc

B, T = 8, 40
C, H, W = 3, 192, 192
NC, NS, L = 2, 16, 16
NW = NC * NS
HH = H // 2
NCH = 2 * C
ROW_UNROLL = W // L
SC_NUM, SC_DEN = 8, 32


def _split_point(nv):
    return (nv * SC_NUM) // SC_DEN


def _sc_chunk_sum(xbuf, ybuf):
    def step(r, acc):
        for u in range(ROW_UNROLL):
            xv = xbuf[r, pl.ds(u * L, L)]
            yv = ybuf[r, pl.ds(u * L, L)]
            acc = acc + jnp.abs(xv - yv)
        return acc

    return lax.fori_loop(0, HH, step, jnp.zeros((L,), jnp.float32))


@functools.partial(
    pl.kernel,
    out_type=jax.ShapeDtypeStruct((NW, L), jnp.float32),
    mesh=plsc.VectorSubcoreMesh(
        core_axis_name="c", subcore_axis_name="s", num_cores=NC, num_subcores=NS
    ),
    compiler_params=pltpu.CompilerParams(needs_layout_passes=False),
    scratch_types=[
        pltpu.VMEM((HH, W), jnp.float32),
        pltpu.VMEM((HH, W), jnp.float32),
        pltpu.VMEM((HH, W), jnp.float32),
        pltpu.VMEM((HH, W), jnp.float32),
        pltpu.VMEM((16,), jnp.int32),
        pltpu.VMEM((L,), jnp.float32),
        pltpu.SemaphoreType.DMA,
        pltpu.SemaphoreType.DMA,
        pltpu.SemaphoreType.DMA,
        pltpu.SemaphoreType.DMA,
    ],
)
def _sc_l1(x_hbm, y_hbm, vl_hbm, out_hbm,
           xb0, xb1, yb0, yb1, vlv, accv, sx0, sx1, sy0, sy1):
    cid = lax.axis_index("c")
    sid = lax.axis_index("s")
    wid = sid * NC + cid

    pltpu.sync_copy(vl_hbm, vlv.at[pl.ds(0, B)])
    iota = lax.iota(jnp.int32, 16)
    vl = jnp.where(iota < B, vlv[...], 0)
    cum = plsc.cumsum(vl)
    cumex = cum - vl
    nv = jnp.max(cum)
    n_sc = _split_point(nv)

    nf = (n_sc - wid + (NW - 1)) // NW
    nf = jnp.maximum(nf, 0)
    nq = nf * NCH

    def chunk_loc(q):
        k = q // NCH
        c = q - k * NCH
        j = wid + NW * k
        bb = jnp.sum((cum <= j).astype(jnp.int32))
        start = jnp.sum(jnp.where(iota == bb, cumex, 0))
        t = j - start
        ch = c // 2
        h0 = (c - 2 * ch) * HH
        return bb, t, ch, h0

    def start_q(q, xbuf, ybuf, sx, sy):
        bb, t, ch, h0 = chunk_loc(q)
        pltpu.async_copy(x_hbm.at[bb, t, ch, pl.ds(h0, HH), :], xbuf, sx)
        pltpu.async_copy(y_hbm.at[bb, t, ch, pl.ds(h0, HH), :], ybuf, sy)

    def wait_q(xbuf, ybuf, sx, sy):
        pltpu.make_async_copy(x_hbm.at[0, 0, 0, pl.ds(0, HH), :], xbuf, sx).wait()
        pltpu.make_async_copy(y_hbm.at[0, 0, 0, pl.ds(0, HH), :], ybuf, sy).wait()

    @pl.when(nq > 0)
    def _():
        start_q(0, xb0, yb0, sx0, sy0)

    @pl.when(nq > 1)
    def _():
        start_q(1, xb1, yb1, sx1, sy1)

    def pair(g, acc):
        q0 = 2 * g
        q1 = q0 + 1
        wait_q(xb0, yb0, sx0, sy0)
        acc = acc + _sc_chunk_sum(xb0, yb0)

        @pl.when(q0 + 2 < nq)
        def _():
            start_q(q0 + 2, xb0, yb0, sx0, sy0)

        @pl.when(q1 < nq)
        def _():
            wait_q(xb1, yb1, sx1, sy1)

        s1 = _sc_chunk_sum(xb1, yb1)
        acc = acc + jnp.where(q1 < nq, s1, 0.0)

        @pl.when(q1 + 2 < nq)
        def _():
            start_q(q1 + 2, xb1, yb1, sx1, sy1)

        return acc

    acc = lax.fori_loop(0, (nq + 1) // 2, pair, jnp.zeros((L,), jnp.float32))
    accv[...] = acc
    pltpu.sync_copy(accv, out_hbm.at[wid])


def _tc_body(vl_ref, x_hbm, y_hbm, out_ref, *bufs_and_sems):
    nb = TC_NBUF
    xbs = bufs_and_sems[0:nb]
    ybs = bufs_and_sems[nb:2 * nb]
    sxs = bufs_and_sems[2 * nb:3 * nb]
    sys_ = bufs_and_sems[3 * nb:4 * nb]
    def cum_scan(i, carry):
        nv, _ = carry
        return nv + vl_ref[i], 0

    nv, _ = lax.fori_loop(0, B, cum_scan, (0, 0))
    n_sc = _split_point(nv)
    n_tc = nv - n_sc

    def frame_of(j):
        def body(i, carry):
            b, cum, start = carry
            newcum = cum + vl_ref[i]
            take = newcum <= j
            b = jnp.where(take, b + 1, b)
            start = jnp.where(take, newcum, start)
            return b, newcum, start

        b, _, start = lax.fori_loop(0, B, body, (0, 0, 0))
        return b, j - start

    def start_q(q, xbuf, ybuf, sx, sy):
        b, t = frame_of(n_sc + q)
        pltpu.make_async_copy(x_hbm.at[b, t], xbuf, sx).start()
        pltpu.make_async_copy(y_hbm.at[b, t], ybuf, sy).start()

    def wait_q(xbuf, ybuf, sx, sy):
        pltpu.make_async_copy(x_hbm.at[0, 0], xbuf, sx).wait()
        pltpu.make_async_copy(y_hbm.at[0, 0], ybuf, sy).wait()

    for k in range(nb):
        @pl.when(n_tc > k)
        def _(k=k):
            start_q(k, xbs[k], ybs[k], sxs[k], sys_[k])

    def plane_sum(xbuf, ybuf):
        d = jnp.abs(xbuf[...] - ybuf[...])
        return d[0] + d[1] + d[2]

    def group(g, acc):
        qbase = nb * g
        wait_q(xbs[0], ybs[0], sxs[0], sys_[0])
        acc = acc + plane_sum(xbs[0], ybs[0])

        @pl.when(qbase + nb < n_tc)
        def _():
            start_q(qbase + nb, xbs[0], ybs[0], sxs[0], sys_[0])

        for k in range(1, nb):
            q = qbase + k

            @pl.when(q < n_tc)
            def _(k=k):
                wait_q(xbs[k], ybs[k], sxs[k], sys_[k])

            sk = plane_sum(xbs[k], ybs[k])
            acc = acc + jnp.where(q < n_tc, sk, jnp.zeros((H, W), jnp.float32))

            @pl.when(q + nb < n_tc)
            def _(k=k, q=q):
                start_q(q + nb, xbs[k], ybs[k], sxs[k], sys_[k])

        return acc

    acc = lax.fori_loop(0, (n_tc + nb - 1) // nb, group,
                        jnp.zeros((H, W), jnp.float32))
    out_ref[0, 0] = jnp.sum(acc)


TC_NBUF = 6

_tc_l1 = pl.pallas_call(
    _tc_body,
    grid=(1,),
    in_specs=[
        pl.BlockSpec(memory_space=pltpu.SMEM),
        pl.BlockSpec(memory_space=pltpu.HBM),
        pl.BlockSpec(memory_space=pltpu.HBM),
    ],
    out_specs=pl.BlockSpec(memory_space=pltpu.SMEM),
    out_shape=jax.ShapeDtypeStruct((1, 1), jnp.float32),
    scratch_shapes=(
        [pltpu.VMEM((C, H, W), jnp.float32) for _ in range(2 * TC_NBUF)]
        + [pltpu.SemaphoreType.DMA for _ in range(2 * TC_NBUF)]
    ),
)


def kernel(inputs, gt, valid_len):
    vl32 = valid_len.astype(jnp.int32)
    sc_partials = _sc_l1(inputs, gt, vl32)
    tc_partial = _tc_l1(vl32, inputs, gt)
    total = jnp.sum(sc_partials) + tc_partial[0, 0]
    count = jnp.sum(valid_len).astype(inputs.dtype) * (C * H * W)
    return total / count


# speedup vs baseline: 1.0299x; 1.0059x over previous
import functools

import jax
import jax.numpy as jnp
from jax import lax
from jax.experimental import pallas as pl
from jax.experimental.pallas import tpu as pltpu
from jax.experimental.pallas import tpu_sc as plsc

B, T = 8, 40
C, H, W = 3, 192, 192
NC, NS, L = 2, 16, 16
NW = NC * NS
HH = H // 2
NCH = 2 * C
ROW_UNROLL = W // L
SC_NUM, SC_DEN = 8, 32


def _split_point(nv):
    return (nv * SC_NUM) // SC_DEN


def _sc_chunk_sum(xbuf, ybuf):
    def step(r, acc):
        for u in range(ROW_UNROLL):
            xv = xbuf[r, pl.ds(u * L, L)]
            yv = ybuf[r, pl.ds(u * L, L)]
            acc = acc + jnp.abs(xv - yv)
        return acc

    return lax.fori_loop(0, HH, step, jnp.zeros((L,), jnp.float32))


@functools.partial(
    pl.kernel,
    out_type=jax.ShapeDtypeStruct((NW, L), jnp.float32),
    mesh=plsc.VectorSubcoreMesh(
        core_axis_name="c", subcore_axis_name="s", num_cores=NC, num_subcores=NS
    ),
    compiler_params=pltpu.CompilerParams(needs_layout_passes=False),
    scratch_types=[
        pltpu.VMEM((HH, W), jnp.float32),
        pltpu.VMEM((HH, W), jnp.float32),
        pltpu.VMEM((HH, W), jnp.float32),
        pltpu.VMEM((HH, W), jnp.float32),
        pltpu.VMEM((16,), jnp.int32),
        pltpu.VMEM((L,), jnp.float32),
        pltpu.SemaphoreType.DMA,
        pltpu.SemaphoreType.DMA,
        pltpu.SemaphoreType.DMA,
        pltpu.SemaphoreType.DMA,
    ],
)
def _sc_l1(x_hbm, y_hbm, vl_hbm, out_hbm,
           xb0, xb1, yb0, yb1, vlv, accv, sx0, sx1, sy0, sy1):
    cid = lax.axis_index("c")
    sid = lax.axis_index("s")
    wid = sid * NC + cid

    pltpu.sync_copy(vl_hbm, vlv.at[pl.ds(0, B)])
    iota = lax.iota(jnp.int32, 16)
    vl = jnp.where(iota < B, vlv[...], 0)
    cum = plsc.cumsum(vl)
    cumex = cum - vl
    nv = jnp.max(cum)
    n_sc = _split_point(nv)

    nf = (n_sc - wid + (NW - 1)) // NW
    nf = jnp.maximum(nf, 0)
    nq = nf * NCH

    def chunk_loc(q):
        k = q // NCH
        c = q - k * NCH
        j = wid + NW * k
        bb = jnp.sum((cum <= j).astype(jnp.int32))
        start = jnp.sum(jnp.where(iota == bb, cumex, 0))
        t = j - start
        ch = c // 2
        h0 = (c - 2 * ch) * HH
        return bb, t, ch, h0

    def start_q(q, xbuf, ybuf, sx, sy):
        bb, t, ch, h0 = chunk_loc(q)
        pltpu.async_copy(x_hbm.at[bb, t, ch, pl.ds(h0, HH), :], xbuf, sx)
        pltpu.async_copy(y_hbm.at[bb, t, ch, pl.ds(h0, HH), :], ybuf, sy)

    def wait_q(xbuf, ybuf, sx, sy):
        pltpu.make_async_copy(x_hbm.at[0, 0, 0, pl.ds(0, HH), :], xbuf, sx).wait()
        pltpu.make_async_copy(y_hbm.at[0, 0, 0, pl.ds(0, HH), :], ybuf, sy).wait()

    @pl.when(nq > 0)
    def _():
        start_q(0, xb0, yb0, sx0, sy0)

    @pl.when(nq > 1)
    def _():
        start_q(1, xb1, yb1, sx1, sy1)

    def pair(g, acc):
        q0 = 2 * g
        q1 = q0 + 1
        wait_q(xb0, yb0, sx0, sy0)
        acc = acc + _sc_chunk_sum(xb0, yb0)

        @pl.when(q0 + 2 < nq)
        def _():
            start_q(q0 + 2, xb0, yb0, sx0, sy0)

        @pl.when(q1 < nq)
        def _():
            wait_q(xb1, yb1, sx1, sy1)

        s1 = _sc_chunk_sum(xb1, yb1)
        acc = acc + jnp.where(q1 < nq, s1, 0.0)

        @pl.when(q1 + 2 < nq)
        def _():
            start_q(q1 + 2, xb1, yb1, sx1, sy1)

        return acc

    acc = lax.fori_loop(0, (nq + 1) // 2, pair, jnp.zeros((L,), jnp.float32))
    accv[...] = acc
    pltpu.sync_copy(accv, out_hbm.at[wid])


def _tc_body(vl_ref, x_hbm, y_hbm, out_ref, *bufs_and_sems):
    nb = TC_NBUF
    xbs = bufs_and_sems[0:nb]
    ybs = bufs_and_sems[nb:2 * nb]
    sxs = bufs_and_sems[2 * nb:3 * nb]
    sys_ = bufs_and_sems[3 * nb:4 * nb]
    def cum_scan(i, carry):
        nv, _ = carry
        return nv + vl_ref[i], 0

    nv, _ = lax.fori_loop(0, B, cum_scan, (0, 0))
    n_sc = _split_point(nv)
    n_tc = nv - n_sc

    def frame_of(j):
        def body(i, carry):
            b, cum, start = carry
            newcum = cum + vl_ref[i]
            take = newcum <= j
            b = jnp.where(take, b + 1, b)
            start = jnp.where(take, newcum, start)
            return b, newcum, start

        b, _, start = lax.fori_loop(0, B, body, (0, 0, 0))
        return b, j - start

    def start_q(q, xbuf, ybuf, sx, sy):
        b, t = frame_of(n_sc + q)
        pltpu.make_async_copy(x_hbm.at[b, t], xbuf, sx).start()
        pltpu.make_async_copy(y_hbm.at[b, t], ybuf, sy).start()

    def wait_q(xbuf, ybuf, sx, sy):
        pltpu.make_async_copy(x_hbm.at[0, 0], xbuf, sx).wait()
        pltpu.make_async_copy(y_hbm.at[0, 0], ybuf, sy).wait()

    for k in range(nb):
        @pl.when(n_tc > k)
        def _(k=k):
            start_q(k, xbs[k], ybs[k], sxs[k], sys_[k])

    def plane_sum(xbuf, ybuf):
        d = jnp.abs(xbuf[...] - ybuf[...])
        return d[0] + d[1] + d[2]

    def group(g, acc):
        qbase = nb * g
        wait_q(xbs[0], ybs[0], sxs[0], sys_[0])
        acc = acc + plane_sum(xbs[0], ybs[0])

        @pl.when(qbase + nb < n_tc)
        def _():
            start_q(qbase + nb, xbs[0], ybs[0], sxs[0], sys_[0])

        for k in range(1, nb):
            q = qbase + k

            @pl.when(q < n_tc)
            def _(k=k):
                wait_q(xbs[k], ybs[k], sxs[k], sys_[k])

            sk = plane_sum(xbs[k], ybs[k])
            acc = acc + jnp.where(q < n_tc, sk, jnp.zeros((H, W), jnp.float32))

            @pl.when(q + nb < n_tc)
            def _(k=k, q=q):
                start_q(q + nb, xbs[k], ybs[k], sxs[k], sys_[k])

        return acc

    acc = lax.fori_loop(0, (n_tc + nb - 1) // nb, group,
                        jnp.zeros((H, W), jnp.float32))
    out_ref[0, 0] = jnp.sum(acc)


TC_NBUF = 8

_tc_l1 = pl.pallas_call(
    _tc_body,
    grid=(1,),
    in_specs=[
        pl.BlockSpec(memory_space=pltpu.SMEM),
        pl.BlockSpec(memory_space=pltpu.HBM),
        pl.BlockSpec(memory_space=pltpu.HBM),
    ],
    out_specs=pl.BlockSpec(memory_space=pltpu.SMEM),
    out_shape=jax.ShapeDtypeStruct((1, 1), jnp.float32),
    scratch_shapes=(
        [pltpu.VMEM((C, H, W), jnp.float32) for _ in range(2 * TC_NBUF)]
        + [pltpu.SemaphoreType.DMA for _ in range(2 * TC_NBUF)]
    ),
)


def kernel(inputs, gt, valid_len):
    vl32 = valid_len.astype(jnp.int32)
    sc_partials = _sc_l1(inputs, gt, vl32)
    tc_partial = _tc_l1(vl32, inputs, gt)
    total = jnp.sum(sc_partials) + tc_partial[0, 0]
    count = jnp.sum(valid_len).astype(inputs.dtype) * (C * H * W)
    return total / count


# speedup vs baseline: 1.0310x; 1.0011x over previous
import functools

import jax
import jax.numpy as jnp
from jax import lax
from jax.experimental import pallas as pl
from jax.experimental.pallas import tpu as pltpu
from jax.experimental.pallas import tpu_sc as plsc

B, T = 8, 40
C, H, W = 3, 192, 192
NC, NS, L = 2, 16, 16
NW = NC * NS
HH = H // 2
NCH = 2 * C
ROW_UNROLL = W // L
SC_NUM, SC_DEN = 9, 32


def _split_point(nv):
    return (nv * SC_NUM) // SC_DEN


def _sc_chunk_sum(xbuf, ybuf):
    def step(r, acc):
        for u in range(ROW_UNROLL):
            xv = xbuf[r, pl.ds(u * L, L)]
            yv = ybuf[r, pl.ds(u * L, L)]
            acc = acc + jnp.abs(xv - yv)
        return acc

    return lax.fori_loop(0, HH, step, jnp.zeros((L,), jnp.float32))


@functools.partial(
    pl.kernel,
    out_type=jax.ShapeDtypeStruct((NW, L), jnp.float32),
    mesh=plsc.VectorSubcoreMesh(
        core_axis_name="c", subcore_axis_name="s", num_cores=NC, num_subcores=NS
    ),
    compiler_params=pltpu.CompilerParams(needs_layout_passes=False),
    scratch_types=[
        pltpu.VMEM((HH, W), jnp.float32),
        pltpu.VMEM((HH, W), jnp.float32),
        pltpu.VMEM((HH, W), jnp.float32),
        pltpu.VMEM((HH, W), jnp.float32),
        pltpu.VMEM((16,), jnp.int32),
        pltpu.VMEM((L,), jnp.float32),
        pltpu.SemaphoreType.DMA,
        pltpu.SemaphoreType.DMA,
        pltpu.SemaphoreType.DMA,
        pltpu.SemaphoreType.DMA,
    ],
)
def _sc_l1(x_hbm, y_hbm, vl_hbm, out_hbm,
           xb0, xb1, yb0, yb1, vlv, accv, sx0, sx1, sy0, sy1):
    cid = lax.axis_index("c")
    sid = lax.axis_index("s")
    wid = sid * NC + cid

    pltpu.sync_copy(vl_hbm, vlv.at[pl.ds(0, B)])
    iota = lax.iota(jnp.int32, 16)
    vl = jnp.where(iota < B, vlv[...], 0)
    cum = plsc.cumsum(vl)
    cumex = cum - vl
    nv = jnp.max(cum)
    n_sc = _split_point(nv)

    nf = (n_sc - wid + (NW - 1)) // NW
    nf = jnp.maximum(nf, 0)
    nq = nf * NCH

    def chunk_loc(q):
        k = q // NCH
        c = q - k * NCH
        j = wid + NW * k
        bb = jnp.sum((cum <= j).astype(jnp.int32))
        start = jnp.sum(jnp.where(iota == bb, cumex, 0))
        t = j - start
        ch = c // 2
        h0 = (c - 2 * ch) * HH
        return bb, t, ch, h0

    def start_q(q, xbuf, ybuf, sx, sy):
        bb, t, ch, h0 = chunk_loc(q)
        pltpu.async_copy(x_hbm.at[bb, t, ch, pl.ds(h0, HH), :], xbuf, sx)
        pltpu.async_copy(y_hbm.at[bb, t, ch, pl.ds(h0, HH), :], ybuf, sy)

    def wait_q(xbuf, ybuf, sx, sy):
        pltpu.make_async_copy(x_hbm.at[0, 0, 0, pl.ds(0, HH), :], xbuf, sx).wait()
        pltpu.make_async_copy(y_hbm.at[0, 0, 0, pl.ds(0, HH), :], ybuf, sy).wait()

    @pl.when(nq > 0)
    def _():
        start_q(0, xb0, yb0, sx0, sy0)

    @pl.when(nq > 1)
    def _():
        start_q(1, xb1, yb1, sx1, sy1)

    def pair(g, acc):
        q0 = 2 * g
        q1 = q0 + 1
        wait_q(xb0, yb0, sx0, sy0)
        acc = acc + _sc_chunk_sum(xb0, yb0)

        @pl.when(q0 + 2 < nq)
        def _():
            start_q(q0 + 2, xb0, yb0, sx0, sy0)

        @pl.when(q1 < nq)
        def _():
            wait_q(xb1, yb1, sx1, sy1)

        s1 = _sc_chunk_sum(xb1, yb1)
        acc = acc + jnp.where(q1 < nq, s1, 0.0)

        @pl.when(q1 + 2 < nq)
        def _():
            start_q(q1 + 2, xb1, yb1, sx1, sy1)

        return acc

    acc = lax.fori_loop(0, (nq + 1) // 2, pair, jnp.zeros((L,), jnp.float32))
    accv[...] = acc
    pltpu.sync_copy(accv, out_hbm.at[wid])


def _tc_body(vl_ref, x_hbm, y_hbm, out_ref, *bufs_and_sems):
    nb = TC_NBUF
    xbs = bufs_and_sems[0:nb]
    ybs = bufs_and_sems[nb:2 * nb]
    sxs = bufs_and_sems[2 * nb:3 * nb]
    sys_ = bufs_and_sems[3 * nb:4 * nb]
    def cum_scan(i, carry):
        nv, _ = carry
        return nv + vl_ref[i], 0

    nv, _ = lax.fori_loop(0, B, cum_scan, (0, 0))
    n_sc = _split_point(nv)
    n_tc = nv - n_sc

    def frame_of(j):
        def body(i, carry):
            b, cum, start = carry
            newcum = cum + vl_ref[i]
            take = newcum <= j
            b = jnp.where(take, b + 1, b)
            start = jnp.where(take, newcum, start)
            return b, newcum, start

        b, _, start = lax.fori_loop(0, B, body, (0, 0, 0))
        return b, j - start

    def start_q(q, xbuf, ybuf, sx, sy):
        b, t = frame_of(n_sc + q)
        pltpu.make_async_copy(x_hbm.at[b, t], xbuf, sx).start()
        pltpu.make_async_copy(y_hbm.at[b, t], ybuf, sy).start()

    def wait_q(xbuf, ybuf, sx, sy):
        pltpu.make_async_copy(x_hbm.at[0, 0], xbuf, sx).wait()
        pltpu.make_async_copy(y_hbm.at[0, 0], ybuf, sy).wait()

    for k in range(nb):
        @pl.when(n_tc > k)
        def _(k=k):
            start_q(k, xbs[k], ybs[k], sxs[k], sys_[k])

    def plane_sum(xbuf, ybuf):
        d = jnp.abs(xbuf[...] - ybuf[...])
        return d[0] + d[1] + d[2]

    def group(g, acc):
        qbase = nb * g
        wait_q(xbs[0], ybs[0], sxs[0], sys_[0])
        acc = acc + plane_sum(xbs[0], ybs[0])

        @pl.when(qbase + nb < n_tc)
        def _():
            start_q(qbase + nb, xbs[0], ybs[0], sxs[0], sys_[0])

        for k in range(1, nb):
            q = qbase + k

            @pl.when(q < n_tc)
            def _(k=k):
                wait_q(xbs[k], ybs[k], sxs[k], sys_[k])

            sk = plane_sum(xbs[k], ybs[k])
            acc = acc + jnp.where(q < n_tc, sk, jnp.zeros((H, W), jnp.float32))

            @pl.when(q + nb < n_tc)
            def _(k=k, q=q):
                start_q(q + nb, xbs[k], ybs[k], sxs[k], sys_[k])

        return acc

    acc = lax.fori_loop(0, (n_tc + nb - 1) // nb, group,
                        jnp.zeros((H, W), jnp.float32))
    out_ref[0, 0] = jnp.sum(acc)


TC_NBUF = 8

_tc_l1 = pl.pallas_call(
    _tc_body,
    grid=(1,),
    in_specs=[
        pl.BlockSpec(memory_space=pltpu.SMEM),
        pl.BlockSpec(memory_space=pltpu.HBM),
        pl.BlockSpec(memory_space=pltpu.HBM),
    ],
    out_specs=pl.BlockSpec(memory_space=pltpu.SMEM),
    out_shape=jax.ShapeDtypeStruct((1, 1), jnp.float32),
    scratch_shapes=(
        [pltpu.VMEM((C, H, W), jnp.float32) for _ in range(2 * TC_NBUF)]
        + [pltpu.SemaphoreType.DMA for _ in range(2 * TC_NBUF)]
    ),
)


def kernel(inputs, gt, valid_len):
    vl32 = valid_len.astype(jnp.int32)
    sc_partials = _sc_l1(inputs, gt, vl32)
    tc_partial = _tc_l1(vl32, inputs, gt)
    total = jnp.sum(sc_partials) + tc_partial[0, 0]
    count = jnp.sum(valid_len).astype(inputs.dtype) * (C * H * W)
    return total / count
